# baseline (speedup 1.0000x reference)
import functools
import math

import jax
import jax.numpy as jnp
from jax import lax
from jax.experimental import pallas as pl
from jax.experimental.pallas import tpu as pltpu
from jax.experimental.pallas import tpu_sc as plsc

B, L, D, VOCAB = 1024, 200, 128, 100000
HALF = D // 2
CH = 128
NCHUNK = B * L // CH
NW = 32
CPW = NCHUNK // NW
CYC = 25
NBUF = 5
DEPTH = NBUF - 1


def _bf16_bits(x):
    u = lax.bitcast_convert_type(x, jnp.uint32)
    return (u + 0x7FFF + ((u >> 16) & 1)) >> 16


def _trig_body(trig_ref):
    row = lax.broadcasted_iota(jnp.int32, (L // 2, D), 0)
    lane = lax.broadcasted_iota(jnp.int32, (L // 2, D), 1)
    pos = (2 * row + lane // HALF).astype(jnp.float32)
    fi = lax.rem(lane, HALF).astype(jnp.float32)
    ang = pos * jnp.exp(fi * (-math.log(10000.0) / D))
    packed = _bf16_bits(jnp.cos(ang)) | (_bf16_bits(jnp.sin(ang)) << 16)
    trig_ref[...] = packed.astype(jnp.int32)


def _make_tables():
    return pl.pallas_call(
        _trig_body,
        out_shape=jax.ShapeDtypeStruct((L // 2, D), jnp.int32),
    )()


@functools.partial(
    pl.kernel,
    mesh=plsc.VectorSubcoreMesh(core_axis_name="c", subcore_axis_name="s"),
    out_type=jax.ShapeDtypeStruct((NCHUNK, CH, D), jnp.float32),
    scratch_types=[
        pltpu.VMEM((CPW * CH,), jnp.int32),
        pltpu.VMEM((NBUF, CH, D), jnp.float32),
        pltpu.VMEM((L // 2, D), jnp.int32),
        pltpu.SemaphoreType.DMA,
        pltpu.SemaphoreType.DMA,
    ],
)
def _sc_rope_gather(x_hbm, table_hbm, trig_hbm, out_hbm,
                    idx_v, rows_v, trig_v, gsem, osem):
    wid = lax.axis_index("s") * 2 + lax.axis_index("c")
    pltpu.sync_copy(trig_hbm, trig_v)
    base = wid * CPW
    pltpu.sync_copy(x_hbm.at[wid], idx_v)

    def fire_gather(k, slot):
        pltpu.make_async_copy(
            table_hbm.at[idx_v.at[pl.ds(k * CH, CH)]],
            rows_v.at[slot], gsem).start()

    def wait_gather(slot):
        pltpu.make_async_copy(
            table_hbm.at[idx_v.at[pl.ds(0, CH)]],
            rows_v.at[slot], gsem).wait()

    def fire_out(k, slot):
        pltpu.make_async_copy(
            rows_v.at[slot], out_hbm.at[base + k], osem).start()

    def wait_out(k, slot):
        pltpu.make_async_copy(
            rows_v.at[slot], out_hbm.at[base + k], osem).wait()

    def rope_rows(p, lo, hi, off):
        def row_body(rr, inner):
            pos = rr + off
            for j in range(HALF // 16):
                e = rows_v[p, rr, pl.ds(j * 16, 16)]
                o = rows_v[p, rr, pl.ds(HALF + j * 16, 16)]
                w = trig_v[pos // 2, pl.ds((pos % 2) * HALF + j * 16, 16)]
                cv = lax.bitcast_convert_type(w << 16, jnp.float32)
                sv = lax.bitcast_convert_type(w, jnp.float32)
                rows_v[p, rr, pl.ds(j * 16, 16)] = e * cv - o * sv
                rows_v[p, rr, pl.ds(HALF + j * 16, 16)] = e * sv + o * cv
            return inner

        lax.fori_loop(lo, hi, row_body, 0)

    def compute(t, p):
        pbase = (CH * t) % L
        split = min(L - pbase, CH)
        rope_rows(p, 0, split, pbase)
        if split < CH:
            rope_rows(p, split, CH, pbase - L)

    def step(k, t, p):
        wait_gather(p)
        compute(t, p)
        fire_out(k, p)
        if t == 0:
            pl.when(k >= 1)(lambda: wait_out(k - 1, (p + DEPTH) % NBUF))
        else:
            wait_out(k - 1, (p + DEPTH) % NBUF)
        if t + DEPTH < CYC:
            fire_gather(k + DEPTH, (p + DEPTH) % NBUF)
        else:
            pl.when(k + DEPTH < CPW)(
                lambda: fire_gather(k + DEPTH, (p + DEPTH) % NBUF))

    for s in range(DEPTH):
        fire_gather(s, s)

    def cycle_body(g, carry):
        for t in range(CYC):
            step(g * CYC + t, t, t % NBUF)
        return carry

    lax.fori_loop(0, CPW // CYC, cycle_body, 0)
    wait_out(CPW - 1, (CPW - 1) % NBUF)


def kernel(x, table):
    x = x.reshape(NW, CPW * CH).astype(jnp.int32)
    table = table.astype(jnp.float32)
    trig_t = _make_tables()
    out = _sc_rope_gather(x, table, trig_t)
    return out.reshape(B, L, D)


# speedup vs baseline: 2.0724x; 2.0724x over previous
import functools
import math

import jax
import jax.numpy as jnp
from jax import lax
from jax.experimental import pallas as pl
from jax.experimental.pallas import tpu as pltpu
from jax.experimental.pallas import tpu_sc as plsc

B, L, D, VOCAB = 1024, 200, 128, 100000
HALF = D // 2
CH = 128
NCHUNK = B * L // CH
NW = 32
CPW = NCHUNK // NW
CYC = 25
NBUF = 5
DEPTH = NBUF - 1


def _bf16_bits(x):
    u = lax.bitcast_convert_type(x, jnp.uint32)
    return (u + 0x7FFF + ((u >> 16) & 1)) >> 16


def _trig_body(trig_ref):
    pos = lax.broadcasted_iota(jnp.int32, (L, D), 0).astype(jnp.float32)
    lane = lax.broadcasted_iota(jnp.int32, (L, D), 1)
    fi = lax.rem(lane, HALF).astype(jnp.float32)
    ang = pos * jnp.exp(fi * (-math.log(10000.0) / D))
    packed = _bf16_bits(jnp.cos(ang)) | (_bf16_bits(jnp.sin(ang)) << 16)
    trig_ref[...] = jnp.where(lane < HALF, packed, 0).astype(jnp.int32)


def _make_tables():
    return pl.pallas_call(
        _trig_body,
        out_shape=jax.ShapeDtypeStruct((L, D), jnp.int32),
    )()


@functools.partial(
    pl.kernel,
    mesh=plsc.VectorSubcoreMesh(core_axis_name="c", subcore_axis_name="s"),
    out_type=jax.ShapeDtypeStruct((NCHUNK, CH, D), jnp.float32),
    scratch_types=[
        pltpu.VMEM((CPW * CH,), jnp.int32),
        pltpu.VMEM((NBUF, CH, D), jnp.float32),
        pltpu.VMEM((L, D), jnp.int32),
        pltpu.SemaphoreType.DMA,
        pltpu.SemaphoreType.DMA,
    ],
)
def _sc_rope_gather(x_hbm, table_hbm, trig_hbm, out_hbm,
                    idx_v, rows_v, trig_v, gsem, osem):
    wid = lax.axis_index("s") * 2 + lax.axis_index("c")
    pltpu.sync_copy(trig_hbm, trig_v)
    base = wid * CPW
    pltpu.sync_copy(x_hbm.at[wid], idx_v)

    def fire_gather(k, slot):
        pltpu.make_async_copy(
            table_hbm.at[idx_v.at[pl.ds(k * CH, CH)]],
            rows_v.at[slot], gsem).start()

    def wait_gather(slot):
        pltpu.make_async_copy(
            table_hbm.at[idx_v.at[pl.ds(0, CH)]],
            rows_v.at[slot], gsem).wait()

    def fire_out(k, slot):
        pltpu.make_async_copy(
            rows_v.at[slot], out_hbm.at[base + k], osem).start()

    def wait_out(k, slot):
        pltpu.make_async_copy(
            rows_v.at[slot], out_hbm.at[base + k], osem).wait()

    def rope_rows(p, lo, hi, off):
        def row_body(rr, inner):
            pos = rr + off
            for j in range(HALF // 16):
                e = rows_v[p, rr, pl.ds(j * 16, 16)]
                o = rows_v[p, rr, pl.ds(HALF + j * 16, 16)]
                w = trig_v[pos, pl.ds(j * 16, 16)]
                cv = lax.bitcast_convert_type(w << 16, jnp.float32)
                sv = lax.bitcast_convert_type(w, jnp.float32)
                rows_v[p, rr, pl.ds(j * 16, 16)] = e * cv - o * sv
                rows_v[p, rr, pl.ds(HALF + j * 16, 16)] = e * sv + o * cv
            return inner

        lax.fori_loop(lo, hi, row_body, 0)

    def compute(t, p):
        pbase = (CH * t) % L
        split = min(L - pbase, CH)
        rope_rows(p, 0, split, pbase)
        if split < CH:
            rope_rows(p, split, CH, pbase - L)

    def step(k, t, p):
        wait_gather(p)
        compute(t, p)
        fire_out(k, p)
        if t == 0:
            pl.when(k >= 1)(lambda: wait_out(k - 1, (p + DEPTH) % NBUF))
        else:
            wait_out(k - 1, (p + DEPTH) % NBUF)
        if t + DEPTH < CYC:
            fire_gather(k + DEPTH, (p + DEPTH) % NBUF)
        else:
            pl.when(k + DEPTH < CPW)(
                lambda: fire_gather(k + DEPTH, (p + DEPTH) % NBUF))

    for s in range(DEPTH):
        fire_gather(s, s)

    def cycle_body(g, carry):
        for t in range(CYC):
            step(g * CYC + t, t, t % NBUF)
        return carry

    lax.fori_loop(0, CPW // CYC, cycle_body, 0)
    wait_out(CPW - 1, (CPW - 1) % NBUF)


def kernel(x, table):
    x = x.reshape(NW, CPW * CH).astype(jnp.int32)
    table = table.astype(jnp.float32)
    trig_t = _make_tables()
    out = _sc_rope_gather(x, table, trig_t)
    return out.reshape(B, L, D)


# speedup vs baseline: 2.1181x; 1.0220x over previous
import functools
import math

import jax
import jax.numpy as jnp
from jax import lax
from jax.experimental import pallas as pl
from jax.experimental.pallas import tpu as pltpu
from jax.experimental.pallas import tpu_sc as plsc

B, L, D, VOCAB = 1024, 200, 128, 100000
HALF = D // 2
CH = 80
NCHUNK = B * L // CH
NW = 32
CPW = NCHUNK // NW
CYC = 5
NBUF = 5
DEPTH = NBUF - 1


def _bf16_bits(x):
    u = lax.bitcast_convert_type(x, jnp.uint32)
    return (u + 0x7FFF + ((u >> 16) & 1)) >> 16


def _trig_body(trig_ref):
    pos = lax.broadcasted_iota(jnp.int32, (L, D), 0).astype(jnp.float32)
    lane = lax.broadcasted_iota(jnp.int32, (L, D), 1)
    fi = lax.rem(lane, HALF).astype(jnp.float32)
    ang = pos * jnp.exp(fi * (-math.log(10000.0) / D))
    packed = _bf16_bits(jnp.cos(ang)) | (_bf16_bits(jnp.sin(ang)) << 16)
    trig_ref[...] = jnp.where(lane < HALF, packed, 0).astype(jnp.int32)


def _make_tables():
    return pl.pallas_call(
        _trig_body,
        out_shape=jax.ShapeDtypeStruct((L, D), jnp.int32),
    )()


@functools.partial(
    pl.kernel,
    mesh=plsc.VectorSubcoreMesh(core_axis_name="c", subcore_axis_name="s"),
    out_type=jax.ShapeDtypeStruct((NCHUNK, CH, D), jnp.float32),
    scratch_types=[
        pltpu.VMEM((CPW * CH,), jnp.int32),
        pltpu.VMEM((NBUF, CH, D), jnp.float32),
        pltpu.VMEM((L, D), jnp.int32),
        pltpu.SemaphoreType.DMA,
        pltpu.SemaphoreType.DMA,
    ],
)
def _sc_rope_gather(x_hbm, table_hbm, trig_hbm, out_hbm,
                    idx_v, rows_v, trig_v, gsem, osem):
    wid = lax.axis_index("s") * 2 + lax.axis_index("c")
    pltpu.sync_copy(trig_hbm, trig_v)
    base = wid * CPW
    pltpu.sync_copy(x_hbm.at[wid], idx_v)

    def fire_gather(k, slot):
        pltpu.make_async_copy(
            table_hbm.at[idx_v.at[pl.ds(k * CH, CH)]],
            rows_v.at[slot], gsem).start()

    def wait_gather(slot):
        pltpu.make_async_copy(
            table_hbm.at[idx_v.at[pl.ds(0, CH)]],
            rows_v.at[slot], gsem).wait()

    def fire_out(k, slot):
        pltpu.make_async_copy(
            rows_v.at[slot], out_hbm.at[base + k], osem).start()

    def wait_out(k, slot):
        pltpu.make_async_copy(
            rows_v.at[slot], out_hbm.at[base + k], osem).wait()

    def rope_rows(p, lo, hi, off):
        def row_body(rr, inner):
            pos = rr + off
            for j in range(HALF // 16):
                e = rows_v[p, rr, pl.ds(j * 16, 16)]
                o = rows_v[p, rr, pl.ds(HALF + j * 16, 16)]
                w = trig_v[pos, pl.ds(j * 16, 16)]
                cv = lax.bitcast_convert_type(w << 16, jnp.float32)
                sv = lax.bitcast_convert_type(w, jnp.float32)
                rows_v[p, rr, pl.ds(j * 16, 16)] = e * cv - o * sv
                rows_v[p, rr, pl.ds(HALF + j * 16, 16)] = e * sv + o * cv
            return inner

        lax.fori_loop(lo, hi, row_body, 0)

    def compute(t, p):
        pbase = (CH * t) % L
        split = min(L - pbase, CH)
        rope_rows(p, 0, split, pbase)
        if split < CH:
            rope_rows(p, split, CH, pbase - L)

    def step(k, t, p):
        wait_gather(p)
        compute(t, p)
        fire_out(k, p)
        if t == 0:
            pl.when(k >= 1)(lambda: wait_out(k - 1, (p + DEPTH) % NBUF))
        else:
            wait_out(k - 1, (p + DEPTH) % NBUF)
        if t + DEPTH < CYC:
            fire_gather(k + DEPTH, (p + DEPTH) % NBUF)
        else:
            pl.when(k + DEPTH < CPW)(
                lambda: fire_gather(k + DEPTH, (p + DEPTH) % NBUF))

    for s in range(DEPTH):
        fire_gather(s, s)

    def cycle_body(g, carry):
        for t in range(CYC):
            step(g * CYC + t, t, t % NBUF)
        return carry

    lax.fori_loop(0, CPW // CYC, cycle_body, 0)
    wait_out(CPW - 1, (CPW - 1) % NBUF)


def kernel(x, table):
    x = x.reshape(NW, CPW * CH).astype(jnp.int32)
    table = table.astype(jnp.float32)
    trig_t = _make_tables()
    out = _sc_rope_gather(x, table, trig_t)
    return out.reshape(B, L, D)
